# Initial kernel scaffold; baseline (speedup 1.0000x reference)
import jax
import jax.numpy as jnp
from jax.experimental import pallas as pl


def kernel(indices, a, b):
    raise NotImplementedError("write your pallas kernel here")


# speedup vs baseline: 1.0523x; 1.0523x over previous
import functools

import jax
import jax.numpy as jnp
from jax import lax
from jax.experimental import pallas as pl
from jax.experimental.pallas import tpu as pltpu
from jax.experimental.pallas import tpu_sc as plsc

N = 32768
L = 16

_info = plsc.get_sparse_core_info()
_NC, _NS = _info.num_cores, _info.num_subcores
_NW = _NC * _NS
_BPW = N // _NW


def _body(idx_hbm, a_hbm, b_hbm, out_a_hbm, out_b_hbm,
          a_v, b_v, idx_v, oa_v, ob_v):
    wid = lax.axis_index("s") * _NC + lax.axis_index("c")
    base = wid * _BPW
    pltpu.sync_copy(a_hbm, a_v)
    pltpu.sync_copy(b_hbm, b_v)
    pltpu.sync_copy(idx_hbm.at[pl.ds(base, _BPW)], idx_v)

    def step(i, carry):
        off = pl.multiple_of(i * L, L)
        ii = idx_v[pl.ds(off, L)]
        oa_v[pl.ds(off, L)] = plsc.load_gather(a_v, [ii])
        ob_v[pl.ds(off, L)] = plsc.load_gather(b_v, [ii])
        return carry

    lax.fori_loop(0, _BPW // L, step, 0)
    pltpu.sync_copy(oa_v, out_a_hbm.at[pl.ds(base, _BPW)])
    pltpu.sync_copy(ob_v, out_b_hbm.at[pl.ds(base, _BPW)])


@jax.jit
def kernel(indices, a, b):
    idx = indices.astype(jnp.int32)
    f32 = jnp.float32
    call = pl.kernel(
        _body,
        mesh=plsc.VectorSubcoreMesh(core_axis_name="c", subcore_axis_name="s"),
        compiler_params=pltpu.CompilerParams(needs_layout_passes=False),
        out_type=(
            jax.ShapeDtypeStruct((N,), f32),
            jax.ShapeDtypeStruct((N,), f32),
        ),
        scratch_types=[
            pltpu.VMEM((N,), f32),
            pltpu.VMEM((N,), f32),
            pltpu.VMEM((_BPW,), jnp.int32),
            pltpu.VMEM((_BPW,), f32),
            pltpu.VMEM((_BPW,), f32),
        ],
    )
    return call(idx, a, b)


# speedup vs baseline: 1.3666x; 1.2986x over previous
import jax
import jax.numpy as jnp
from jax import lax
from jax.experimental import pallas as pl
from jax.experimental.pallas import tpu as pltpu
from jax.experimental.pallas import tpu_sc as plsc

N = 32768

_info = plsc.get_sparse_core_info()
_NC, _NS = _info.num_cores, _info.num_subcores
_NW = _NC * _NS
_BPW = N // _NW
_CH = 128
_NCH = _BPW // _CH
_SEG = N // _NS


def _body(idx_hbm, a_hbm, b_hbm, out_a_hbm, out_b_hbm,
          sh_a, sh_b, idx_v, oa_v, ob_v, sem):
    cid = lax.axis_index("c")
    sid = lax.axis_index("s")
    wid = sid * _NC + cid
    seg = sid * _SEG
    pltpu.sync_copy(a_hbm.at[pl.ds(seg, _SEG)], sh_a.at[pl.ds(seg, _SEG)])
    pltpu.sync_copy(b_hbm.at[pl.ds(seg, _SEG)], sh_b.at[pl.ds(seg, _SEG)])
    pltpu.sync_copy(idx_hbm.at[wid], idx_v)
    plsc.subcore_barrier()
    copies = []
    for j in range(_NCH):
        copies.append(pltpu.async_copy(sh_a.at[idx_v.at[j]], oa_v.at[j], sem))
        copies.append(pltpu.async_copy(sh_b.at[idx_v.at[j]], ob_v.at[j], sem))
    for c in copies:
        c.wait()
    pltpu.sync_copy(oa_v, out_a_hbm.at[wid])
    pltpu.sync_copy(ob_v, out_b_hbm.at[wid])


@jax.jit
def kernel(indices, a, b):
    idx = indices.astype(jnp.int32).reshape(_NW, _NCH, _CH)
    f32 = jnp.float32
    call = pl.kernel(
        _body,
        mesh=plsc.VectorSubcoreMesh(core_axis_name="c", subcore_axis_name="s"),
        compiler_params=pltpu.CompilerParams(needs_layout_passes=False),
        out_type=(
            jax.ShapeDtypeStruct((_NW, _NCH, _CH), f32),
            jax.ShapeDtypeStruct((_NW, _NCH, _CH), f32),
        ),
        scratch_types=[
            pltpu.VMEM_SHARED((N,), f32),
            pltpu.VMEM_SHARED((N,), f32),
            pltpu.VMEM((_NCH, _CH), jnp.int32),
            pltpu.VMEM((_NCH, _CH), f32),
            pltpu.VMEM((_NCH, _CH), f32),
            pltpu.SemaphoreType.DMA,
        ],
    )
    out_a, out_b = call(idx, a, b)
    return out_a.reshape(N), out_b.reshape(N)
